# baseline (speedup 1.0000x reference)
import functools

import jax
import jax.numpy as jnp
from jax.experimental import pallas as pl
from jax.experimental.pallas import tpu as pltpu

N = 10000
E = 160000
F = 128
D = 128
C = 16
M = 50000
K = 8
HOPS = 3
RETRIEVE_W = 0.5
LABEL_W = 0.5

_TN = 1000
_TM = 2048
_MPAD = 51200


def _extract_topk(v, i, k):
    big = jnp.int32(2**31 - 1)
    neg = jnp.float32(-jnp.inf)
    outs_v, outs_i = [], []
    for _ in range(k):
        mv = jnp.max(v, axis=1, keepdims=True)
        ismax = v == mv
        mi = jnp.min(jnp.where(ismax, i, big), axis=1, keepdims=True)
        outs_v.append(mv)
        outs_i.append(mi)
        v = jnp.where(i == mi, neg, v)
    return jnp.concatenate(outs_v, axis=1), jnp.concatenate(outs_i, axis=1)


def _knn_body(pre_ref, base_ref, vals_ref, idx_ref, *, k, tm):
    j = pl.program_id(1)
    sims = jax.lax.dot_general(
        pre_ref[...], base_ref[...],
        (((1,), (0,)), ((), ())),
        preferred_element_type=jnp.float32,
    )
    tn = sims.shape[0]
    nseg = min(128, tm)
    depth = tm // nseg
    slices = [sims[:, t * nseg:(t + 1) * nseg] for t in range(depth)]
    smax = slices[0]
    for t in range(1, depth):
        smax = jnp.maximum(smax, slices[t])
    scol = jax.lax.broadcasted_iota(jnp.int32, (tn, nseg), 1)
    _, spos = _extract_topk(smax, scol, k)
    cand = jnp.concatenate(
        [jnp.take_along_axis(sl, spos, axis=1) for sl in slices], axis=1)
    cand_pos = jnp.concatenate(
        [spos + t * nseg for t in range(depth)], axis=1)
    tv, ti = _extract_topk(cand, cand_pos + j * tm, k)

    @pl.when(j == 0)
    def _():
        vals_ref[...] = tv
        idx_ref[...] = ti

    @pl.when(j > 0)
    def _():
        wv = jnp.concatenate([vals_ref[...], tv], axis=1)
        wi = jnp.concatenate([idx_ref[...], ti], axis=1)
        nv, ni = _extract_topk(wv, wi, k)
        vals_ref[...] = nv
        idx_ref[...] = ni


def _knn_topk(pre, base_emb_t, tn, tm, interpret=False):
    n, d = pre.shape
    m = base_emb_t.shape[1]
    return pl.pallas_call(
        functools.partial(_knn_body, k=K, tm=tm),
        grid=(n // tn, m // tm),
        in_specs=[
            pl.BlockSpec((tn, d), lambda i, j: (i, 0)),
            pl.BlockSpec((d, tm), lambda i, j: (0, j)),
        ],
        out_specs=[
            pl.BlockSpec((tn, K), lambda i, j: (i, 0)),
            pl.BlockSpec((tn, K), lambda i, j: (i, 0)),
        ],
        out_shape=[
            jax.ShapeDtypeStruct((n, K), jnp.float32),
            jax.ShapeDtypeStruct((n, K), jnp.int32),
        ],
        compiler_params=pltpu.CompilerParams(
            dimension_semantics=("parallel", "arbitrary")),
        interpret=interpret,
    )(pre, base_emb_t)


def kernel(features, edge_index, W_enc, base_emb, base_labels, W1, b1, W2, b2):
    src = edge_index[0]
    dst = edge_index[1]
    deg = jnp.clip(jnp.zeros((N,), dtype=jnp.float32).at[dst].add(1.0), 1.0, None)

    h = features @ W_enc
    pre = jax.nn.relu(
        jax.ops.segment_sum(h[src], dst, num_segments=N) / deg[:, None])

    base_pad_t = jnp.concatenate(
        [base_emb.T, jnp.zeros((D, _MPAD - M), dtype=base_emb.dtype)], axis=1)
    top_v, top_i = _knn_topk(pre, base_pad_t, _TN, _TM)
    w = jax.nn.softmax(top_v, axis=1)
    rag_embedding = jnp.einsum("nkd,nk->nd", jnp.take(base_emb, top_i, axis=0), w)
    rag_label = jnp.mean(jnp.take(base_labels, top_i, axis=0), axis=1)

    x = pre
    for _ in range(HOPS):
        x = jax.ops.segment_sum(x[src], dst, num_segments=N) / deg[:, None]

    hidden = x * (1.0 - RETRIEVE_W) + rag_embedding * RETRIEVE_W
    dec = jax.nn.relu(hidden @ W1 + b1) @ W2 + b2
    decode_label = jax.nn.softmax(dec, axis=1)
    return decode_label * (1.0 - LABEL_W) + rag_label * LABEL_W


# speedup vs baseline: 1.2053x; 1.2053x over previous
import functools

import jax
import jax.numpy as jnp
from jax.experimental import pallas as pl
from jax.experimental.pallas import tpu as pltpu

N = 10000
E = 160000
F = 128
D = 128
C = 16
M = 50000
K = 8
HOPS = 3
RETRIEVE_W = 0.5
LABEL_W = 0.5

_TN = 1000
_TM = 2048
_MPAD = 51200


def _extract_topk(v, pos_f, k):
    tn, w = v.shape
    neg = jnp.float32(-jnp.inf)
    vals, masked_pos = [], []
    x = v
    for _ in range(k):
        mv = jnp.max(x, axis=1, keepdims=True)
        mask = x == mv
        vals.append(mv)
        masked_pos.append(jnp.where(mask, pos_f, 0.0))
        x = jnp.where(mask, neg, x)
    p = jnp.concatenate(masked_pos, axis=1)
    r = jax.lax.broadcasted_iota(jnp.int32, (k * w, k), 0) // w
    c = jax.lax.broadcasted_iota(jnp.int32, (k * w, k), 1)
    bd = (r == c).astype(jnp.float32)
    pos = jax.lax.dot_general(p, bd, (((1,), (0,)), ((), ())),
                              preferred_element_type=jnp.float32)
    return jnp.concatenate(vals, axis=1), pos


def _knn_body(pre_ref, base_ref, vals_ref, idx_ref, *, k, tm):
    j = pl.program_id(1)
    sims = jax.lax.dot_general(
        pre_ref[...], base_ref[...],
        (((1,), (0,)), ((), ())),
        preferred_element_type=jnp.float32,
    )
    tn = sims.shape[0]
    nseg = min(128, tm)
    depth = tm // nseg
    slices = [sims[:, t * nseg:(t + 1) * nseg] for t in range(depth)]
    smax = slices[0]
    for t in range(1, depth):
        smax = jnp.maximum(smax, slices[t])
    scol_f = jax.lax.broadcasted_iota(
        jnp.int32, (tn, nseg), 1).astype(jnp.float32)
    _, spos_f = _extract_topk(smax, scol_f, k)
    spos = spos_f.astype(jnp.int32)
    cand = jnp.concatenate(
        [jnp.take_along_axis(sl, spos, axis=1) for sl in slices], axis=1)
    cand_pos_f = jnp.concatenate(
        [spos_f + t * nseg for t in range(depth)], axis=1) + j * tm
    tv, ti = _extract_topk(cand, cand_pos_f, k)

    @pl.when(j == 0)
    def _():
        vals_ref[...] = tv
        idx_ref[...] = ti

    @pl.when(j > 0)
    def _():
        wv = jnp.concatenate([vals_ref[...], tv], axis=1)
        wi = jnp.concatenate([idx_ref[...], ti], axis=1)
        nv, ni = _extract_topk(wv, wi, k)
        vals_ref[...] = nv
        idx_ref[...] = ni


def _knn_topk(pre, base_emb_t, tn, tm, interpret=False):
    n, d = pre.shape
    m = base_emb_t.shape[1]
    return pl.pallas_call(
        functools.partial(_knn_body, k=K, tm=tm),
        grid=(n // tn, m // tm),
        in_specs=[
            pl.BlockSpec((tn, d), lambda i, j: (i, 0)),
            pl.BlockSpec((d, tm), lambda i, j: (0, j)),
        ],
        out_specs=[
            pl.BlockSpec((tn, K), lambda i, j: (i, 0)),
            pl.BlockSpec((tn, K), lambda i, j: (i, 0)),
        ],
        out_shape=[
            jax.ShapeDtypeStruct((n, K), jnp.float32),
            jax.ShapeDtypeStruct((n, K), jnp.float32),
        ],
        compiler_params=pltpu.CompilerParams(
            dimension_semantics=("parallel", "arbitrary")),
        interpret=interpret,
    )(pre, base_emb_t)


def kernel(features, edge_index, W_enc, base_emb, base_labels, W1, b1, W2, b2):
    src = edge_index[0]
    dst = edge_index[1]
    deg = jnp.clip(jnp.zeros((N,), dtype=jnp.float32).at[dst].add(1.0), 1.0, None)

    h = features @ W_enc
    pre = jax.nn.relu(
        jax.ops.segment_sum(h[src], dst, num_segments=N) / deg[:, None])

    base_pad_t = jnp.concatenate(
        [base_emb.T, jnp.zeros((D, _MPAD - M), dtype=base_emb.dtype)], axis=1)
    top_v, top_if = _knn_topk(pre, base_pad_t, _TN, _TM)
    top_i = top_if.astype(jnp.int32)
    w = jax.nn.softmax(top_v, axis=1)
    rag_embedding = jnp.einsum("nkd,nk->nd", jnp.take(base_emb, top_i, axis=0), w)
    rag_label = jnp.mean(jnp.take(base_labels, top_i, axis=0), axis=1)

    x = pre
    for _ in range(HOPS):
        x = jax.ops.segment_sum(x[src], dst, num_segments=N) / deg[:, None]

    hidden = x * (1.0 - RETRIEVE_W) + rag_embedding * RETRIEVE_W
    dec = jax.nn.relu(hidden @ W1 + b1) @ W2 + b2
    decode_label = jax.nn.softmax(dec, axis=1)
    return decode_label * (1.0 - LABEL_W) + rag_label * LABEL_W
